# baseline (speedup 1.0000x reference)
import functools
import math

import jax
import jax.numpy as jnp
from jax import lax
from jax.experimental import pallas as pl
from jax.experimental.pallas import tpu as pltpu
from jax.experimental.pallas import tpu_sc as plsc


def _make_sc_kernel(N, D, scale):
    info = plsc.get_sparse_core_info()
    NC, NS, L = info.num_cores, info.num_subcores, info.num_lanes
    NW = NC * NS
    per_w = N // NW
    C = 32
    GR = 16
    n_chunks = per_w // C
    n_groups = C // GR
    NB = 3
    mesh = plsc.VectorSubcoreMesh(core_axis_name="c", subcore_axis_name="s")

    @functools.partial(
        pl.kernel,
        mesh=mesh,
        out_type=jax.ShapeDtypeStruct((N, D), jnp.float32),
        scratch_types=[
            pltpu.VMEM((n_chunks * n_groups, GR), jnp.int32),
            pltpu.VMEM((C, D), jnp.float32),
            pltpu.VMEM((C, D), jnp.float32),
            pltpu.VMEM((C, D), jnp.float32),
            pltpu.SemaphoreType.DMA((NB, n_groups)),
            pltpu.SemaphoreType.DMA((NB,)),
        ],
    )
    def k(x_hbm, lut_hbm, out_hbm, idx_all, rows0, rows1, rows2, gsem, ssem):
        wid = lax.axis_index("s") * NC + lax.axis_index("c")
        base = wid * per_w
        rows = (rows0, rows1, rows2)

        pltpu.sync_copy(x_hbm.at[wid], idx_all)

        def gather_group(c, g):
            b = c % NB
            return pltpu.async_copy(
                lut_hbm.at[idx_all.at[c * n_groups + g]],
                rows[b].at[pl.ds(g * GR, GR), :],
                gsem.at[b, g])

        def gather(c):
            return [gather_group(c, g) for g in range(n_groups)]

        def scatter_group(c, g):
            b = c % NB
            return pltpu.async_copy(
                rows[b].at[pl.ds(g * GR, GR), :],
                out_hbm.at[pl.ds(base + c * C + g * GR, GR), :],
                ssem.at[b])

        def scale_group(rv, g):
            def row_body(rr, carry):
                r = rr * 2
                for j in range(D // L):
                    sl = pl.ds(j * L, L)
                    rv[r, sl] = rv[r, sl] * scale
                for j in range(D // L):
                    sl = pl.ds(j * L, L)
                    rv[r + 1, sl] = rv[r + 1, sl] * scale
                return carry
            lax.fori_loop(g * GR // 2, (g + 1) * GR // 2, row_body, 0)

        h_g = [None] * n_chunks
        h_s = [[None] * n_groups for _ in range(n_chunks)]
        h_g[0] = gather(0)
        h_g[1] = gather(1)
        for c in range(n_chunks):
            if c + 2 < n_chunks:
                if c - 1 >= 0:
                    for hh in h_s[c - 1]:
                        hh.wait()
                h_g[c + 2] = gather(c + 2)
            for g in range(n_groups):
                h_g[c][g].wait()
                scale_group(rows[c % NB], g)
                h_s[c][g] = scatter_group(c, g)
        for c in (n_chunks - 3, n_chunks - 2, n_chunks - 1):
            for hh in h_s[c]:
                hh.wait()

    return k


def kernel(x, lut):
    B, S = x.shape
    _, D = lut.shape
    N = B * S
    info = plsc.get_sparse_core_info()
    NW = info.num_cores * info.num_subcores
    per_w = N // NW
    GR = 16
    scale = float(math.sqrt(D))
    xf = x.reshape(NW, per_w // GR, GR).astype(jnp.int32)
    out = _make_sc_kernel(N, D, scale)(xf, lut)
    return out.reshape(B, S, D)


# speedup vs baseline: 1.2114x; 1.2114x over previous
import functools
import math

import jax
import jax.numpy as jnp
from jax import lax
from jax.experimental import pallas as pl
from jax.experimental.pallas import tpu as pltpu
from jax.experimental.pallas import tpu_sc as plsc


def _make_sc_kernel(N, D, scale):
    info = plsc.get_sparse_core_info()
    NC, NS, L = info.num_cores, info.num_subcores, info.num_lanes
    NW = NC * NS
    per_w = N // NW
    C = 32
    GR = 8
    n_chunks = per_w // C
    n_groups = C // GR
    NB = 3
    mesh = plsc.VectorSubcoreMesh(core_axis_name="c", subcore_axis_name="s")

    @functools.partial(
        pl.kernel,
        mesh=mesh,
        out_type=jax.ShapeDtypeStruct((N, D), jnp.float32),
        scratch_types=[
            pltpu.VMEM((n_chunks * n_groups, GR), jnp.int32),
            pltpu.VMEM((C, D), jnp.float32),
            pltpu.VMEM((C, D), jnp.float32),
            pltpu.VMEM((C, D), jnp.float32),
            pltpu.SemaphoreType.DMA((NB, n_groups)),
            pltpu.SemaphoreType.DMA((NB,)),
        ],
    )
    def k(x_hbm, lut_hbm, out_hbm, idx_all, rows0, rows1, rows2, gsem, ssem):
        wid = lax.axis_index("s") * NC + lax.axis_index("c")
        base = wid * per_w
        rows = (rows0, rows1, rows2)

        pltpu.sync_copy(x_hbm.at[wid], idx_all)

        def gather_group(c, g):
            b = c % NB
            return pltpu.async_copy(
                lut_hbm.at[idx_all.at[c * n_groups + g]],
                rows[b].at[pl.ds(g * GR, GR), :],
                gsem.at[b, g])

        def gather(c):
            return [gather_group(c, g) for g in range(n_groups)]

        def scatter_group(c, g):
            b = c % NB
            return pltpu.async_copy(
                rows[b].at[pl.ds(g * GR, GR), :],
                out_hbm.at[pl.ds(base + c * C + g * GR, GR), :],
                ssem.at[b])

        def scale_group(rv, g):
            def row_body(r, carry):
                for j in range(D // L):
                    sl = pl.ds(j * L, L)
                    rv[r, sl] = rv[r, sl] * scale
                return carry
            lax.fori_loop(g * GR, (g + 1) * GR, row_body, 0)

        h_g = [None] * n_chunks
        h_s = [[None] * n_groups for _ in range(n_chunks)]
        h_g[0] = gather(0)
        h_g[1] = gather(1)
        for c in range(n_chunks):
            if c + 2 < n_chunks:
                if c - 1 >= 0:
                    for hh in h_s[c - 1]:
                        hh.wait()
                h_g[c + 2] = gather(c + 2)
            for g in range(n_groups):
                h_g[c][g].wait()
                scale_group(rows[c % NB], g)
                h_s[c][g] = scatter_group(c, g)
        for c in (n_chunks - 3, n_chunks - 2, n_chunks - 1):
            for hh in h_s[c]:
                hh.wait()

    return k


def kernel(x, lut):
    B, S = x.shape
    _, D = lut.shape
    N = B * S
    info = plsc.get_sparse_core_info()
    NW = info.num_cores * info.num_subcores
    per_w = N // NW
    GR = 8
    scale = float(math.sqrt(D))
    xf = x.reshape(NW, per_w // GR, GR).astype(jnp.int32)
    out = _make_sc_kernel(N, D, scale)(xf, lut)
    return out.reshape(B, S, D)


# speedup vs baseline: 1.2828x; 1.0589x over previous
import functools
import math

import jax
import jax.numpy as jnp
from jax import lax
from jax.experimental import pallas as pl
from jax.experimental.pallas import tpu as pltpu
from jax.experimental.pallas import tpu_sc as plsc


def _make_sc_kernel(N, D, scale):
    info = plsc.get_sparse_core_info()
    NC, NS, L = info.num_cores, info.num_subcores, info.num_lanes
    NW = NC * NS
    per_w = N // NW
    C = 16
    GR = 16
    n_chunks = per_w // C
    n_groups = C // GR
    NB = 4
    mesh = plsc.VectorSubcoreMesh(core_axis_name="c", subcore_axis_name="s")

    @functools.partial(
        pl.kernel,
        mesh=mesh,
        out_type=jax.ShapeDtypeStruct((N, D), jnp.float32),
        scratch_types=[
            pltpu.VMEM((n_chunks * n_groups, GR), jnp.int32),
            pltpu.VMEM((C, D), jnp.float32),
            pltpu.VMEM((C, D), jnp.float32),
            pltpu.VMEM((C, D), jnp.float32),
            pltpu.VMEM((C, D), jnp.float32),
            pltpu.SemaphoreType.DMA((NB, n_groups)),
            pltpu.SemaphoreType.DMA((NB,)),
        ],
    )
    def k(x_hbm, lut_hbm, out_hbm, idx_all,
          rows0, rows1, rows2, rows3, gsem, ssem):
        wid = lax.axis_index("s") * NC + lax.axis_index("c")
        base = wid * per_w
        rows = (rows0, rows1, rows2, rows3)

        pltpu.sync_copy(x_hbm.at[wid], idx_all)

        def gather_group(c, g):
            b = c % NB
            return pltpu.async_copy(
                lut_hbm.at[idx_all.at[c * n_groups + g]],
                rows[b].at[pl.ds(g * GR, GR), :],
                gsem.at[b, g])

        def gather(c):
            return [gather_group(c, g) for g in range(n_groups)]

        def scatter_group(c, g):
            b = c % NB
            return pltpu.async_copy(
                rows[b].at[pl.ds(g * GR, GR), :],
                out_hbm.at[pl.ds(base + c * C + g * GR, GR), :],
                ssem.at[b])

        def scale_group(rv, g):
            def row_body(r, carry):
                for j in range(D // L):
                    sl = pl.ds(j * L, L)
                    rv[r, sl] = rv[r, sl] * scale
                return carry
            lax.fori_loop(g * GR, (g + 1) * GR, row_body, 0)

        h_g = [None] * n_chunks
        h_s = [[None] * n_groups for _ in range(n_chunks)]
        h_g[0] = gather(0)
        h_g[1] = gather(1)
        for c in range(n_chunks):
            if c + 2 < n_chunks:
                if c - 2 >= 0:
                    for hh in h_s[c - 2]:
                        hh.wait()
                h_g[c + 2] = gather(c + 2)
            for g in range(n_groups):
                h_g[c][g].wait()
                scale_group(rows[c % NB], g)
                h_s[c][g] = scatter_group(c, g)
        for c in range(n_chunks - 4, n_chunks):
            for hh in h_s[c]:
                hh.wait()

    return k


def kernel(x, lut):
    B, S = x.shape
    _, D = lut.shape
    N = B * S
    info = plsc.get_sparse_core_info()
    NW = info.num_cores * info.num_subcores
    per_w = N // NW
    GR = 16
    scale = float(math.sqrt(D))
    xf = x.reshape(NW, per_w // GR, GR).astype(jnp.int32)
    out = _make_sc_kernel(N, D, scale)(xf, lut)
    return out.reshape(B, S, D)
